# baseline (speedup 1.0000x reference)
import functools

import jax
import jax.numpy as jnp
from jax import lax
from jax.experimental import pallas as pl
from jax.experimental.pallas import tpu as pltpu
from jax.experimental.pallas import tpu_sc as plsc

_N = 50000
_E = 800000
_C = 16
_K = 16
_D = 3
_NP = 50176
_CHT = 6272
_GN = _NP // _CHT
_NTASK = 51
_CHE = 3200
_NCH = _E // _CHE

_F32 = jnp.float32


def _erf(x):
    a1, a2, a3, a4, a5 = 0.254829592, -0.284496736, 1.421413741, -1.453152027, 1.061405429
    p = 0.3275911
    s = jnp.sign(x)
    z = jnp.abs(x)
    t = 1.0 / (1.0 + p * z)
    poly = ((((a5 * t + a4) * t + a3) * t + a2) * t + a1) * t
    return s * (1.0 - poly * jnp.exp(-z * z))


def _gelu(x):
    return 0.5 * x * (1.0 + _erf(x * 0.7071067811865476))


def _sc_task(tid, h_hbm, pk_hbm, egw_hbm, out_hbm,
             fn_v, acc_v, pb0, eb0, pb1, eb1, sem0, sem1):
    d = tid // 17
    c = tid - d * 17
    widx = lax.iota(jnp.int32, 16) * 3 + d
    pltpu.sync_copy(h_hbm.at[c], fn_v)

    def zbody(i, carry):
        acc_v[pl.ds(i * 16, 16)] = jnp.zeros((16,), _F32)
        return carry
    lax.fori_loop(0, _NP // 16, zbody, 0)

    def start(ci, pb, eb, sem):
        off = ci * _CHE
        pltpu.async_copy(pk_hbm.at[pl.ds(off, _CHE)], pb, sem)
        pltpu.async_copy(egw_hbm.at[pl.ds(off * 3, 3 * _CHE)], eb, sem)

    def wait2(pb, eb, sem):
        pltpu.make_async_copy(pk_hbm.at[pl.ds(0, _CHE)], pb, sem).wait()
        pltpu.make_async_copy(egw_hbm.at[pl.ds(0, 3 * _CHE)], eb, sem).wait()

    def process(pb, eb):
        @plsc.parallel_loop(0, _CHE, 16, unroll=32)
        def gbody(o):
            pk = pb[pl.ds(o, 16)]
            s = pk & 0xFFFF
            t = lax.shift_right_logical(pk, 16)
            w = plsc.load_gather(eb, [o * 3 + widx])
            vals = plsc.load_gather(fn_v, [s])
            plsc.addupdate_scatter(acc_v, [t], vals * w)

    start(0, pb0, eb0, sem0)
    start(1, pb1, eb1, sem1)

    def cbody(i, carry):
        ci = i * 2
        wait2(pb0, eb0, sem0)
        process(pb0, eb0)

        @pl.when(ci + 2 < _NCH)
        def _():
            start(ci + 2, pb0, eb0, sem0)

        wait2(pb1, eb1, sem1)
        process(pb1, eb1)

        @pl.when(ci + 3 < _NCH)
        def _():
            start(ci + 3, pb1, eb1, sem1)
        return carry
    lax.fori_loop(0, _NCH // 2, cbody, 0)

    pltpu.sync_copy(acc_v, out_hbm.at[tid])


@functools.partial(
    pl.kernel,
    out_type=jax.ShapeDtypeStruct((_NTASK, _NP), _F32),
    mesh=plsc.VectorSubcoreMesh(core_axis_name="c", subcore_axis_name="s",
                                num_cores=2, num_subcores=16),
    compiler_params=pltpu.CompilerParams(needs_layout_passes=False,
                                         use_tc_tiling_on_sc=True),
    scratch_types=[
        pltpu.VMEM((_NP,), _F32),
        pltpu.VMEM((_NP,), _F32),
        pltpu.VMEM((_CHE,), jnp.int32),
        pltpu.VMEM((3 * _CHE,), _F32),
        pltpu.VMEM((_CHE,), jnp.int32),
        pltpu.VMEM((3 * _CHE,), _F32),
        pltpu.SemaphoreType.DMA,
        pltpu.SemaphoreType.DMA,
    ],
)
def _sc_grad(h_hbm, pk_hbm, egw_hbm, out_hbm,
             fn_v, acc_v, pb0, eb0, pb1, eb1, sem0, sem1):
    wid = lax.axis_index("s") * 2 + lax.axis_index("c")
    for p in range(2):
        tid = wid + 32 * p

        @pl.when(tid < _NTASK)
        def _():
            _sc_task(tid, h_hbm, pk_hbm, egw_hbm, out_hbm,
                     fn_v, acc_v, pb0, eb0, pb1, eb1, sem0, sem1)


def _full(shape):
    return pl.BlockSpec(shape, lambda *g: tuple(0 for _ in shape))


def _chunk(rows):
    return pl.BlockSpec((rows, _CHT), lambda *g: (0, g[-1]))


def _k1_body(nodes_ref, x_ref, nwm_ref, mm_ref, fc0w_ref, fc0b_ref,
             bc_ref, bs_ref, wbc_ref, wbs_ref, h_ref):
    t = lax.dot_general(mm_ref[...], nodes_ref[...], (((1,), (0,)), ((), ())),
                        preferred_element_type=_F32)
    bc = jnp.cos(t)
    bs = jnp.sin(t)
    wv = nwm_ref[0:1, :] * nwm_ref[1:2, :]
    bc_ref[...] = bc
    bs_ref[...] = bs
    wbc_ref[...] = bc * wv
    wbs_ref[...] = bs * wv
    h = lax.dot_general(fc0w_ref[...], x_ref[...], (((1,), (0,)), ((), ())),
                        preferred_element_type=_F32) + fc0b_ref[...]
    h_ref[...] = jnp.concatenate(
        [h, jnp.ones((1, _CHT), _F32), jnp.zeros((7, _CHT), _F32)], axis=0)


def _k1(nodes_T, x_T, nwm, mm, fc0_w, fc0_b):
    return pl.pallas_call(
        _k1_body,
        grid=(_GN,),
        in_specs=[_chunk(8), _chunk(8), _chunk(8), _full((16, 8)),
                  _full((16, 8)), _full((16, 1))],
        out_specs=[_chunk(16), _chunk(16), _chunk(16), _chunk(16), _chunk(24)],
        out_shape=[jax.ShapeDtypeStruct((16, _NP), _F32)] * 4
        + [jax.ShapeDtypeStruct((24, _NP), _F32)],
    )(nodes_T, x_T, nwm, mm, fc0_w, fc0_b)


def _k23_body(wc_ref, ws_ref, wsw_ref, wsb_ref, gwsp_ref, gwsd_ref, gwsb_ref,
              w1_ref, b1_ref, w2_ref, b2_ref,
              h_ref, wbc_ref, wbs_ref, bc_ref, bs_ref, a_ref,
              out_ref, sc_s, ss_s, *, last):
    p = pl.program_id(0)
    h = h_ref[0:16, :]

    @pl.when(p == 0)
    def _():
        @pl.when(pl.program_id(1) == 0)
        def _():
            sc_s[...] = jnp.zeros_like(sc_s)
            ss_s[...] = jnp.zeros_like(ss_s)
        sc_s[...] += lax.dot_general(h, wbc_ref[...], (((1,), (1,)), ((), ())),
                                     preferred_element_type=_F32)
        ss_s[...] += lax.dot_general(h, wbs_ref[...], (((1,), (1,)), ((), ())),
                                     preferred_element_type=_F32)

    @pl.when(p == 1)
    def _():
        Sc = sc_s[...]
        Ss = ss_s[...]
        wc = wc_ref[...]
        ws = ws_ref[...]
        ein = lambda X, W: jnp.sum(X[:, None, :] * W, axis=0)
        f_c = ein(Sc, wc) + ein(Ss, ws)
        f_s = ein(Sc, ws) - ein(Ss, wc)
        x1 = 2.0 * (lax.dot_general(f_c, bc_ref[...], (((1,), (0,)), ((), ())),
                                    preferred_element_type=_F32)
                    - lax.dot_general(f_s, bs_ref[...], (((1,), (0,)), ((), ())),
                                      preferred_element_type=_F32))
        x2 = lax.dot_general(wsw_ref[...], h, (((1,), (0,)), ((), ())),
                             preferred_element_type=_F32) + wsb_ref[...]
        x3 = lax.dot_general(gwsp_ref[...], a_ref[...], (((1,), (0,)), ((), ())),
                             preferred_element_type=_F32) + gwsb_ref[...]
        for dd in range(_D):
            corr = lax.dot_general(gwsd_ref[dd], h, (((1,), (0,)), ((), ())),
                                   preferred_element_type=_F32)
            x3 = x3 - corr * a_ref[dd * 17 + 16:dd * 17 + 17, :]
        hn = x1 + x2 + x3
        if last:
            z = _gelu(lax.dot_general(w1_ref[...], hn, (((1,), (0,)), ((), ())),
                                      preferred_element_type=_F32) + b1_ref[...])
            out_ref[...] = lax.dot_general(w2_ref[...], z, (((1,), (0,)), ((), ())),
                                           preferred_element_type=_F32) + b2_ref[...]
        else:
            out_ref[...] = jnp.concatenate(
                [_gelu(hn), jnp.ones((1, _CHT), _F32),
                 jnp.zeros((7, _CHT), _F32)], axis=0)


def _k23(wc, ws, wsw, wsb, gwsp, gwsd, gwsb, w1, b1, w2, b2,
         h_ext, wbc, wbs, bc, bs, A, last):
    out_rows = 1 if last else 24
    return pl.pallas_call(
        functools.partial(_k23_body, last=last),
        grid=(2, _GN),
        in_specs=[_full((16, 16, 16)), _full((16, 16, 16)),
                  _full((16, 16)), _full((16, 1)),
                  _full((16, _NTASK)), _full((3, 16, 16)), _full((16, 1)),
                  _full((128, 16)), _full((128, 1)), _full((1, 128)),
                  _full((1, 1)),
                  _chunk(24), _chunk(16), _chunk(16), _chunk(16), _chunk(16),
                  _chunk(_NTASK)],
        out_specs=_chunk(out_rows),
        out_shape=jax.ShapeDtypeStruct((out_rows, _NP), _F32),
        scratch_shapes=[pltpu.VMEM((16, 16), _F32), pltpu.VMEM((16, 16), _F32)],
    )(wc, ws, wsw, wsb, gwsp, gwsd, gwsb, w1, b1, w2, b2,
      h_ext, wbc, wbs, bc, bs, A)


def kernel(x, node_mask, nodes, node_weights, directed_edges,
           edge_gradient_weights, sp_L, modes, fc0_w, fc0_b, ws_w, ws_b,
           gws_w, gws_b, spec_wc, spec_ws, spec_w0, fc1_w, fc1_b, fc2_w,
           fc2_b):
    pad = _NP - _N
    mm = jnp.pad((modes * sp_L[None, :, :])[:, :, 0], ((0, 0), (0, 5)))
    nodes_T = jnp.pad(nodes[0].T, ((0, 5), (0, pad)))
    x_T = jnp.pad(x[0].T, ((0, 4), (0, pad)))
    nwm = jnp.pad(jnp.concatenate([node_weights[0].T, node_mask[0].T], 0),
                  ((0, 6), (0, pad)))
    fc0_w8 = jnp.pad(fc0_w, ((0, 0), (0, 4)))
    src = directed_edges[0, :, 1]
    tgt = directed_edges[0, :, 0]
    pk = (tgt << 16) | src
    egw_flat = edge_gradient_weights[0].reshape(-1)

    L = ws_w.shape[0]
    g4 = gws_w.reshape(L, _C, _C, _D)
    gws_d = jnp.transpose(g4, (0, 3, 1, 2))
    gp = jnp.transpose(g4, (0, 1, 3, 2))
    gws_p = jnp.concatenate([gp, jnp.zeros((L, _C, _D, 1), _F32)],
                            axis=3).reshape(L, _C, _NTASK)

    bc, bs, wbc, wbs, h_ext = _k1(nodes_T, x_T, nwm, mm, fc0_w8,
                                  fc0_b.reshape(16, 1))

    out = None
    for i in range(L):
        A = _sc_grad(h_ext, pk, egw_flat)
        res = _k23(spec_wc[i][:, :, :, 0], spec_ws[i][:, :, :, 0],
                   ws_w[i], ws_b[i].reshape(16, 1), gws_p[i], gws_d[i],
                   gws_b[i].reshape(16, 1), fc1_w, fc1_b.reshape(128, 1),
                   fc2_w, fc2_b.reshape(1, 1),
                   h_ext, wbc, wbs, bc, bs, A, last=(i == L - 1))
        if i == L - 1:
            out = res
        else:
            h_ext = res

    return out[:, :_N][:, :, None]


# speedup vs baseline: 4.0843x; 4.0843x over previous
import functools

import jax
import jax.numpy as jnp
from jax import lax
from jax.experimental import pallas as pl
from jax.experimental.pallas import tpu as pltpu
from jax.experimental.pallas import tpu_sc as plsc

_N = 50000
_E = 800000
_C = 16
_K = 16
_D = 3
_NP = 50176
_CHT = 6272
_GN = _NP // _CHT
_NTASK = 51
_CHE = 4096
_EP = 802816
_NCH = _EP // _CHE
_PKPAD = -1018167296

_F32 = jnp.float32


def _erf(x):
    a1, a2, a3, a4, a5 = 0.254829592, -0.284496736, 1.421413741, -1.453152027, 1.061405429
    p = 0.3275911
    s = jnp.sign(x)
    z = jnp.abs(x)
    t = 1.0 / (1.0 + p * z)
    poly = ((((a5 * t + a4) * t + a3) * t + a2) * t + a1) * t
    return s * (1.0 - poly * jnp.exp(-z * z))


def _gelu(x):
    return 0.5 * x * (1.0 + _erf(x * 0.7071067811865476))


_P1 = 66


def _sc_unit(tid, piece, clo, chi, h_hbm, pk_hbm, egw_hbm, a_hbm, a2_hbm,
             fn_v, acc_v, pb0, eb0, pb1, eb1, sem0, sem1):
    d = tid // 17
    c = tid - d * 17
    pltpu.sync_copy(h_hbm.at[c], fn_v)

    def zbody(i, carry):
        acc_v[pl.ds(i * 16, 16)] = jnp.zeros((16,), _F32)
        return carry
    lax.fori_loop(0, _NP // 16, zbody, 0)

    ebase = d * _EP

    def start(ci, pb, eb, sem):
        off = ci * _CHE
        pltpu.async_copy(pk_hbm.at[pl.ds(off, _CHE)], pb, sem)
        pltpu.async_copy(egw_hbm.at[pl.ds(ebase + off, _CHE)], eb, sem)

    def wait2(pb, eb, sem):
        pltpu.make_async_copy(pk_hbm.at[pl.ds(0, _CHE)], pb, sem).wait()
        pltpu.make_async_copy(egw_hbm.at[pl.ds(0, _CHE)], eb, sem).wait()

    def process(pb, eb):
        @plsc.parallel_loop(0, _CHE, 16, unroll=32)
        def gbody(o):
            pk = pb[pl.ds(o, 16)]
            s = pk & 0xFFFF
            t = lax.shift_right_logical(pk, 16)
            w = eb[pl.ds(o, 16)]
            vals = plsc.load_gather(fn_v, [s])
            plsc.addupdate_scatter(acc_v, [t], vals * w)

    start(clo, pb0, eb0, sem0)
    start(clo + 1, pb1, eb1, sem1)

    def cbody(i, carry):
        ci = clo + i * 2
        wait2(pb0, eb0, sem0)
        process(pb0, eb0)

        @pl.when(ci + 2 < chi)
        def _():
            start(ci + 2, pb0, eb0, sem0)

        wait2(pb1, eb1, sem1)
        process(pb1, eb1)

        @pl.when(ci + 3 < chi)
        def _():
            start(ci + 3, pb1, eb1, sem1)
        return carry
    lax.fori_loop(0, (chi - clo) // 2, cbody, 0)

    @pl.when(piece == 0)
    def _():
        pltpu.sync_copy(acc_v, a_hbm.at[tid])

    @pl.when(piece == 1)
    def _():
        pltpu.sync_copy(acc_v, a2_hbm.at[tid - 32])

    @pl.when(piece == 2)
    def _():
        pltpu.sync_copy(acc_v, a2_hbm.at[tid - 13])


@functools.partial(
    pl.kernel,
    out_type=[jax.ShapeDtypeStruct((_NTASK, _NP), _F32),
              jax.ShapeDtypeStruct((38, _NP), _F32)],
    mesh=plsc.VectorSubcoreMesh(core_axis_name="c", subcore_axis_name="s",
                                num_cores=2, num_subcores=16),
    compiler_params=pltpu.CompilerParams(needs_layout_passes=False,
                                         use_tc_tiling_on_sc=True),
    scratch_types=[
        pltpu.VMEM((_NP,), _F32),
        pltpu.VMEM((_NP,), _F32),
        pltpu.VMEM((_CHE,), jnp.int32),
        pltpu.VMEM((_CHE,), _F32),
        pltpu.VMEM((_CHE,), jnp.int32),
        pltpu.VMEM((_CHE,), _F32),
        pltpu.SemaphoreType.DMA,
        pltpu.SemaphoreType.DMA,
    ],
)
def _sc_grad(h_hbm, pk_hbm, egw_hbm, a_hbm, a2_hbm,
             fn_v, acc_v, pb0, eb0, pb1, eb1, sem0, sem1):
    wid = lax.axis_index("s") * 2 + lax.axis_index("c")
    rest = [fn_v, acc_v, pb0, eb0, pb1, eb1, sem0, sem1]

    _sc_unit(wid, 0, 0, _NCH, h_hbm, pk_hbm, egw_hbm, a_hbm, a2_hbm, *rest)

    def split_unit(j):
        tid = 32 + j // 3
        piece = j - (j // 3) * 3
        clo = piece * _P1
        chi = jnp.minimum(clo + _P1, _NCH)
        _sc_unit(tid, piece, clo, chi, h_hbm, pk_hbm, egw_hbm, a_hbm, a2_hbm,
                 *rest)

    split_unit(wid)

    @pl.when(wid < 25)
    def _():
        split_unit(wid + 32)


def _full(shape):
    return pl.BlockSpec(shape, lambda *g: tuple(0 for _ in shape))


def _chunk(rows):
    return pl.BlockSpec((rows, _CHT), lambda *g: (0, g[-1]))


def _k1_body(nodes_ref, x_ref, nwm_ref, mm_ref, fc0w_ref, fc0b_ref,
             bc_ref, bs_ref, wbc_ref, wbs_ref, h_ref):
    t = lax.dot_general(mm_ref[...], nodes_ref[...], (((1,), (0,)), ((), ())),
                        preferred_element_type=_F32)
    bc = jnp.cos(t)
    bs = jnp.sin(t)
    wv = nwm_ref[0:1, :] * nwm_ref[1:2, :]
    bc_ref[...] = bc
    bs_ref[...] = bs
    wbc_ref[...] = bc * wv
    wbs_ref[...] = bs * wv
    h = lax.dot_general(fc0w_ref[...], x_ref[...], (((1,), (0,)), ((), ())),
                        preferred_element_type=_F32) + fc0b_ref[...]
    h_ref[...] = jnp.concatenate(
        [h, jnp.ones((1, _CHT), _F32), jnp.zeros((7, _CHT), _F32)], axis=0)


def _k1(nodes_T, x_T, nwm, mm, fc0_w, fc0_b):
    return pl.pallas_call(
        _k1_body,
        grid=(_GN,),
        in_specs=[_chunk(8), _chunk(8), _chunk(8), _full((16, 8)),
                  _full((16, 8)), _full((16, 1))],
        out_specs=[_chunk(16), _chunk(16), _chunk(16), _chunk(16), _chunk(24)],
        out_shape=[jax.ShapeDtypeStruct((16, _NP), _F32)] * 4
        + [jax.ShapeDtypeStruct((24, _NP), _F32)],
    )(nodes_T, x_T, nwm, mm, fc0_w, fc0_b)


def _k23_body(wc_ref, ws_ref, wsw_ref, wsb_ref, gwsp_ref, gwsd_ref, gwsb_ref,
              w1_ref, b1_ref, w2_ref, b2_ref,
              h_ref, wbc_ref, wbs_ref, bc_ref, bs_ref, a_ref, a2_ref,
              out_ref, sc_s, ss_s, *, last):
    p = pl.program_id(0)
    h = h_ref[0:16, :]

    @pl.when(p == 0)
    def _():
        @pl.when(pl.program_id(1) == 0)
        def _():
            sc_s[...] = jnp.zeros_like(sc_s)
            ss_s[...] = jnp.zeros_like(ss_s)
        sc_s[...] += lax.dot_general(h, wbc_ref[...], (((1,), (1,)), ((), ())),
                                     preferred_element_type=_F32)
        ss_s[...] += lax.dot_general(h, wbs_ref[...], (((1,), (1,)), ((), ())),
                                     preferred_element_type=_F32)

    @pl.when(p == 1)
    def _():
        Sc = sc_s[...]
        Ss = ss_s[...]
        wc = wc_ref[...]
        ws = ws_ref[...]
        ein = lambda X, W: jnp.sum(X[:, None, :] * W, axis=0)
        f_c = ein(Sc, wc) + ein(Ss, ws)
        f_s = ein(Sc, ws) - ein(Ss, wc)
        x1 = 2.0 * (lax.dot_general(f_c, bc_ref[...], (((1,), (0,)), ((), ())),
                                    preferred_element_type=_F32)
                    - lax.dot_general(f_s, bs_ref[...], (((1,), (0,)), ((), ())),
                                      preferred_element_type=_F32))
        x2 = lax.dot_general(wsw_ref[...], h, (((1,), (0,)), ((), ())),
                             preferred_element_type=_F32) + wsb_ref[...]
        aeff = jnp.concatenate(
            [a_ref[0:32, :],
             a_ref[32:_NTASK, :] + a2_ref[0:19, :] + a2_ref[19:38, :]],
            axis=0)
        x3 = lax.dot_general(gwsp_ref[...], aeff, (((1,), (0,)), ((), ())),
                             preferred_element_type=_F32) + gwsb_ref[...]
        for dd in range(_D):
            corr = lax.dot_general(gwsd_ref[dd], h, (((1,), (0,)), ((), ())),
                                   preferred_element_type=_F32)
            x3 = x3 - corr * aeff[dd * 17 + 16:dd * 17 + 17, :]
        hn = x1 + x2 + x3
        if last:
            z = _gelu(lax.dot_general(w1_ref[...], hn, (((1,), (0,)), ((), ())),
                                      preferred_element_type=_F32) + b1_ref[...])
            out_ref[...] = lax.dot_general(w2_ref[...], z, (((1,), (0,)), ((), ())),
                                           preferred_element_type=_F32) + b2_ref[...]
        else:
            out_ref[...] = jnp.concatenate(
                [_gelu(hn), jnp.ones((1, _CHT), _F32),
                 jnp.zeros((7, _CHT), _F32)], axis=0)


def _k23(wc, ws, wsw, wsb, gwsp, gwsd, gwsb, w1, b1, w2, b2,
         h_ext, wbc, wbs, bc, bs, A, A2, last):
    out_rows = 1 if last else 24
    return pl.pallas_call(
        functools.partial(_k23_body, last=last),
        grid=(2, _GN),
        in_specs=[_full((16, 16, 16)), _full((16, 16, 16)),
                  _full((16, 16)), _full((16, 1)),
                  _full((16, _NTASK)), _full((3, 16, 16)), _full((16, 1)),
                  _full((128, 16)), _full((128, 1)), _full((1, 128)),
                  _full((1, 1)),
                  _chunk(24), _chunk(16), _chunk(16), _chunk(16), _chunk(16),
                  _chunk(_NTASK), _chunk(38)],
        out_specs=_chunk(out_rows),
        out_shape=jax.ShapeDtypeStruct((out_rows, _NP), _F32),
        scratch_shapes=[pltpu.VMEM((16, 16), _F32), pltpu.VMEM((16, 16), _F32)],
    )(wc, ws, wsw, wsb, gwsp, gwsd, gwsb, w1, b1, w2, b2,
      h_ext, wbc, wbs, bc, bs, A, A2)


def kernel(x, node_mask, nodes, node_weights, directed_edges,
           edge_gradient_weights, sp_L, modes, fc0_w, fc0_b, ws_w, ws_b,
           gws_w, gws_b, spec_wc, spec_ws, spec_w0, fc1_w, fc1_b, fc2_w,
           fc2_b):
    pad = _NP - _N
    mm = jnp.pad((modes * sp_L[None, :, :])[:, :, 0], ((0, 0), (0, 5)))
    nodes_T = jnp.pad(nodes[0].T, ((0, 5), (0, pad)))
    x_T = jnp.pad(x[0].T, ((0, 4), (0, pad)))
    nwm = jnp.pad(jnp.concatenate([node_weights[0].T, node_mask[0].T], 0),
                  ((0, 6), (0, pad)))
    fc0_w8 = jnp.pad(fc0_w, ((0, 0), (0, 4)))
    src = directed_edges[0, :, 1]
    tgt = directed_edges[0, :, 0]
    pk = jnp.pad((tgt << 16) | src, (0, _EP - _E), constant_values=_PKPAD)
    egw_flat = jnp.pad(edge_gradient_weights[0].T,
                       ((0, 0), (0, _EP - _E))).reshape(-1)

    L = ws_w.shape[0]
    g4 = gws_w.reshape(L, _C, _C, _D)
    gws_d = jnp.transpose(g4, (0, 3, 1, 2))
    gp = jnp.transpose(g4, (0, 1, 3, 2))
    gws_p = jnp.concatenate([gp, jnp.zeros((L, _C, _D, 1), _F32)],
                            axis=3).reshape(L, _C, _NTASK)

    bc, bs, wbc, wbs, h_ext = _k1(nodes_T, x_T, nwm, mm, fc0_w8,
                                  fc0_b.reshape(16, 1))

    out = None
    for i in range(L):
        A, A2 = _sc_grad(h_ext, pk, egw_flat)
        res = _k23(spec_wc[i][:, :, :, 0], spec_ws[i][:, :, :, 0],
                   ws_w[i], ws_b[i].reshape(16, 1), gws_p[i], gws_d[i],
                   gws_b[i].reshape(16, 1), fc1_w, fc1_b.reshape(128, 1),
                   fc2_w, fc2_b.reshape(1, 1),
                   h_ext, wbc, wbs, bc, bs, A, A2, last=(i == L - 1))
        if i == L - 1:
            out = res
        else:
            h_ext = res

    return out[:, :_N][:, :, None]


# speedup vs baseline: 4.1926x; 1.0265x over previous
import functools

import jax
import jax.numpy as jnp
from jax import lax
from jax.experimental import pallas as pl
from jax.experimental.pallas import tpu as pltpu
from jax.experimental.pallas import tpu_sc as plsc

_N = 50000
_E = 800000
_C = 16
_K = 16
_D = 3
_NP = 50176
_CHT = 6272
_GN = _NP // _CHT
_NTASK = 51
_CHE = 4096
_EP = 802816
_NCH = _EP // _CHE
_PKPAD = -1018167296

_F32 = jnp.float32


def _erf(x):
    a1, a2, a3, a4, a5 = 0.254829592, -0.284496736, 1.421413741, -1.453152027, 1.061405429
    p = 0.3275911
    s = jnp.sign(x)
    z = jnp.abs(x)
    t = 1.0 / (1.0 + p * z)
    poly = ((((a5 * t + a4) * t + a3) * t + a2) * t + a1) * t
    return s * (1.0 - poly * jnp.exp(-z * z))


def _gelu(x):
    return 0.5 * x * (1.0 + _erf(x * 0.7071067811865476))


_P1 = 66


def _sc_unit(tid, piece, clo, chi, h_hbm, pk_hbm, egw_hbm, a_hbm, a2_hbm,
             fn_v, acc_v, pb0, eb0, pb1, eb1, sem0, sem1):
    d = tid // 17
    c = tid - d * 17
    pltpu.sync_copy(h_hbm.at[c], fn_v)

    def zbody(i, carry):
        acc_v[pl.ds(i * 16, 16)] = jnp.zeros((16,), _F32)
        return carry
    lax.fori_loop(0, _NP // 16, zbody, 0)

    ebase = d * _EP

    def start(ci, pb, eb, sem):
        off = ci * _CHE
        pltpu.async_copy(pk_hbm.at[pl.ds(off, _CHE)], pb, sem)
        pltpu.async_copy(egw_hbm.at[pl.ds(ebase + off, _CHE)], eb, sem)

    def wait2(pb, eb, sem):
        pltpu.make_async_copy(pk_hbm.at[pl.ds(0, _CHE)], pb, sem).wait()
        pltpu.make_async_copy(egw_hbm.at[pl.ds(0, _CHE)], eb, sem).wait()

    def process(pb, eb):
        @plsc.parallel_loop(0, _CHE, 16, unroll=32)
        def gbody(o):
            pk = pb[pl.ds(o, 16)]
            s = pk & 0xFFFF
            t = lax.shift_right_logical(pk, 16)
            w = eb[pl.ds(o, 16)]
            vals = plsc.load_gather(fn_v, [s])
            plsc.addupdate_scatter(acc_v, [t], vals * w)

    start(clo, pb0, eb0, sem0)
    start(clo + 1, pb1, eb1, sem1)

    def cbody(i, carry):
        ci = clo + i * 2
        wait2(pb0, eb0, sem0)
        process(pb0, eb0)

        @pl.when(ci + 2 < chi)
        def _():
            start(ci + 2, pb0, eb0, sem0)

        wait2(pb1, eb1, sem1)
        process(pb1, eb1)

        @pl.when(ci + 3 < chi)
        def _():
            start(ci + 3, pb1, eb1, sem1)
        return carry
    lax.fori_loop(0, (chi - clo) // 2, cbody, 0)

    @pl.when(piece == 0)
    def _():
        pltpu.sync_copy(acc_v, a_hbm.at[tid])

    @pl.when(piece == 1)
    def _():
        pltpu.sync_copy(acc_v, a2_hbm.at[tid - 32])

    @pl.when(piece == 2)
    def _():
        pltpu.sync_copy(acc_v, a2_hbm.at[tid - 13])


@functools.partial(
    pl.kernel,
    out_type=[jax.ShapeDtypeStruct((_NTASK, _NP), _F32),
              jax.ShapeDtypeStruct((38, _NP), _F32)],
    mesh=plsc.VectorSubcoreMesh(core_axis_name="c", subcore_axis_name="s",
                                num_cores=2, num_subcores=16),
    compiler_params=pltpu.CompilerParams(needs_layout_passes=False,
                                         use_tc_tiling_on_sc=True),
    scratch_types=[
        pltpu.VMEM((_NP,), _F32),
        pltpu.VMEM((_NP,), _F32),
        pltpu.VMEM((_CHE,), jnp.int32),
        pltpu.VMEM((_CHE,), _F32),
        pltpu.VMEM((_CHE,), jnp.int32),
        pltpu.VMEM((_CHE,), _F32),
        pltpu.SemaphoreType.DMA,
        pltpu.SemaphoreType.DMA,
    ],
)
def _sc_grad(h_hbm, pk_hbm, egw_hbm, a_hbm, a2_hbm,
             fn_v, acc_v, pb0, eb0, pb1, eb1, sem0, sem1):
    wid = lax.axis_index("s") * 2 + lax.axis_index("c")
    rest = [fn_v, acc_v, pb0, eb0, pb1, eb1, sem0, sem1]

    _sc_unit(wid, 0, 0, _NCH, h_hbm, pk_hbm, egw_hbm, a_hbm, a2_hbm, *rest)

    def split_unit(j):
        tid = 32 + j // 3
        piece = j - (j // 3) * 3
        clo = piece * _P1
        chi = jnp.minimum(clo + _P1, _NCH)
        _sc_unit(tid, piece, clo, chi, h_hbm, pk_hbm, egw_hbm, a_hbm, a2_hbm,
                 *rest)

    split_unit(wid)

    @pl.when(wid < 25)
    def _():
        split_unit(wid + 32)


def _full(shape):
    return pl.BlockSpec(shape, lambda *g: tuple(0 for _ in shape))


def _chunk(rows):
    return pl.BlockSpec((rows, _CHT), lambda *g: (0, g[-1]))


def _k1_body(nodes_ref, x_ref, nwm_ref, mm_ref, fc0w_ref, fc0b_ref,
             bc_ref, bs_ref, wbc_ref, wbs_ref, h_ref):
    t = lax.dot_general(mm_ref[...], nodes_ref[...], (((1,), (0,)), ((), ())),
                        preferred_element_type=_F32)
    bc = jnp.cos(t)
    bs = jnp.sin(t)
    wv = nwm_ref[0:1, :] * nwm_ref[1:2, :]
    bc_ref[...] = bc
    bs_ref[...] = bs
    wbc_ref[...] = bc * wv
    wbs_ref[...] = bs * wv
    h = lax.dot_general(fc0w_ref[...], x_ref[...], (((1,), (0,)), ((), ())),
                        preferred_element_type=_F32) + fc0b_ref[...]
    h_ref[...] = jnp.concatenate(
        [h, jnp.ones((1, _CHT), _F32), jnp.zeros((7, _CHT), _F32)], axis=0)


def _k1(nodes_T, x_T, nwm, mm, fc0_w, fc0_b):
    return pl.pallas_call(
        _k1_body,
        grid=(_GN,),
        in_specs=[_chunk(8), _chunk(8), _chunk(8), _full((16, 8)),
                  _full((16, 8)), _full((16, 1))],
        out_specs=[_chunk(16), _chunk(16), _chunk(16), _chunk(16), _chunk(24)],
        out_shape=[jax.ShapeDtypeStruct((16, _NP), _F32)] * 4
        + [jax.ShapeDtypeStruct((24, _NP), _F32)],
    )(nodes_T, x_T, nwm, mm, fc0_w, fc0_b)


def _k2_body(h_ref, wbc_ref, wbs_ref, sc_ref, ss_ref):
    @pl.when(pl.program_id(0) == 0)
    def _():
        sc_ref[...] = jnp.zeros_like(sc_ref)
        ss_ref[...] = jnp.zeros_like(ss_ref)
    h = h_ref[0:16, :]
    sc_ref[...] += lax.dot_general(h, wbc_ref[...], (((1,), (1,)), ((), ())),
                                   preferred_element_type=_F32)
    ss_ref[...] += lax.dot_general(h, wbs_ref[...], (((1,), (1,)), ((), ())),
                                   preferred_element_type=_F32)


def _k2(h_ext, wbc, wbs):
    return pl.pallas_call(
        _k2_body,
        grid=(_GN,),
        in_specs=[_chunk(24), _chunk(16), _chunk(16)],
        out_specs=[pl.BlockSpec((16, 16), lambda j: (0, 0))] * 2,
        out_shape=[jax.ShapeDtypeStruct((16, 16), _F32)] * 2,
    )(h_ext, wbc, wbs)


def _k3_body(sc_ref, ss_ref, wc_ref, ws_ref, wsw_ref, wsb_ref,
             gwsp_ref, gwsd_ref, gwsb_ref, w1_ref, b1_ref, w2_ref, b2_ref,
             h_ref, bc_ref, bs_ref, a_ref, a2_ref, out_ref, *, last):
    h = h_ref[0:16, :]
    Sc = sc_ref[...]
    Ss = ss_ref[...]
    wc = wc_ref[...]
    ws = ws_ref[...]
    ein = lambda X, W: jnp.sum(X[:, None, :] * W, axis=0)
    f_c = ein(Sc, wc) + ein(Ss, ws)
    f_s = ein(Sc, ws) - ein(Ss, wc)
    x1 = 2.0 * (lax.dot_general(f_c, bc_ref[...], (((1,), (0,)), ((), ())),
                                preferred_element_type=_F32)
                - lax.dot_general(f_s, bs_ref[...], (((1,), (0,)), ((), ())),
                                  preferred_element_type=_F32))
    x2 = lax.dot_general(wsw_ref[...], h, (((1,), (0,)), ((), ())),
                         preferred_element_type=_F32) + wsb_ref[...]
    aeff = jnp.concatenate(
        [a_ref[0:32, :],
         a_ref[32:_NTASK, :] + a2_ref[0:19, :] + a2_ref[19:38, :]],
        axis=0)
    x3 = lax.dot_general(gwsp_ref[...], aeff, (((1,), (0,)), ((), ())),
                         preferred_element_type=_F32) + gwsb_ref[...]
    for dd in range(_D):
        corr = lax.dot_general(gwsd_ref[dd], h, (((1,), (0,)), ((), ())),
                               preferred_element_type=_F32)
        x3 = x3 - corr * aeff[dd * 17 + 16:dd * 17 + 17, :]
    hn = x1 + x2 + x3
    if last:
        z = _gelu(lax.dot_general(w1_ref[...], hn, (((1,), (0,)), ((), ())),
                                  preferred_element_type=_F32) + b1_ref[...])
        out_ref[...] = lax.dot_general(w2_ref[...], z, (((1,), (0,)), ((), ())),
                                       preferred_element_type=_F32) + b2_ref[...]
    else:
        out_ref[...] = jnp.concatenate(
            [_gelu(hn), jnp.ones((1, _CHT), _F32),
             jnp.zeros((7, _CHT), _F32)], axis=0)


def _k3(Sc, Ss, wc, ws, wsw, wsb, gwsp, gwsd, gwsb, w1, b1, w2, b2,
        h_ext, bc, bs, A, A2, last):
    out_rows = 1 if last else 24
    return pl.pallas_call(
        functools.partial(_k3_body, last=last),
        grid=(_GN,),
        in_specs=[_full((16, 16)), _full((16, 16)),
                  _full((16, 16, 16)), _full((16, 16, 16)),
                  _full((16, 16)), _full((16, 1)),
                  _full((16, _NTASK)), _full((3, 16, 16)), _full((16, 1)),
                  _full((128, 16)), _full((128, 1)), _full((1, 128)),
                  _full((1, 1)),
                  _chunk(24), _chunk(16), _chunk(16),
                  _chunk(_NTASK), _chunk(38)],
        out_specs=_chunk(out_rows),
        out_shape=jax.ShapeDtypeStruct((out_rows, _NP), _F32),
    )(Sc, Ss, wc, ws, wsw, wsb, gwsp, gwsd, gwsb, w1, b1, w2, b2,
      h_ext, bc, bs, A, A2)


def kernel(x, node_mask, nodes, node_weights, directed_edges,
           edge_gradient_weights, sp_L, modes, fc0_w, fc0_b, ws_w, ws_b,
           gws_w, gws_b, spec_wc, spec_ws, spec_w0, fc1_w, fc1_b, fc2_w,
           fc2_b):
    pad = _NP - _N
    mm = jnp.pad((modes * sp_L[None, :, :])[:, :, 0], ((0, 0), (0, 5)))
    nodes_T = jnp.pad(nodes[0].T, ((0, 5), (0, pad)))
    x_T = jnp.pad(x[0].T, ((0, 4), (0, pad)))
    nwm = jnp.pad(jnp.concatenate([node_weights[0].T, node_mask[0].T], 0),
                  ((0, 6), (0, pad)))
    fc0_w8 = jnp.pad(fc0_w, ((0, 0), (0, 4)))
    src = directed_edges[0, :, 1]
    tgt = directed_edges[0, :, 0]
    pk = jnp.pad((tgt << 16) | src, (0, _EP - _E), constant_values=_PKPAD)
    egw_flat = jnp.pad(edge_gradient_weights[0].T,
                       ((0, 0), (0, _EP - _E))).reshape(-1)

    L = ws_w.shape[0]
    g4 = gws_w.reshape(L, _C, _C, _D)
    gws_d = jnp.transpose(g4, (0, 3, 1, 2))
    gp = jnp.transpose(g4, (0, 1, 3, 2))
    gws_p = jnp.concatenate([gp, jnp.zeros((L, _C, _D, 1), _F32)],
                            axis=3).reshape(L, _C, _NTASK)

    bc, bs, wbc, wbs, h_ext = _k1(nodes_T, x_T, nwm, mm, fc0_w8,
                                  fc0_b.reshape(16, 1))

    out = None
    for i in range(L):
        A, A2 = _sc_grad(h_ext, pk, egw_flat)
        Sc, Ss = _k2(h_ext, wbc, wbs)
        res = _k3(Sc, Ss, spec_wc[i][:, :, :, 0], spec_ws[i][:, :, :, 0],
                  ws_w[i], ws_b[i].reshape(16, 1), gws_p[i], gws_d[i],
                  gws_b[i].reshape(16, 1), fc1_w, fc1_b.reshape(128, 1),
                  fc2_w, fc2_b.reshape(1, 1),
                  h_ext, bc, bs, A, A2, last=(i == L - 1))
        if i == L - 1:
            out = res
        else:
            h_ext = res

    return out[:, :_N][:, :, None]


# speedup vs baseline: 4.2431x; 1.0120x over previous
import functools

import jax
import jax.numpy as jnp
from jax import lax
from jax.experimental import pallas as pl
from jax.experimental.pallas import tpu as pltpu
from jax.experimental.pallas import tpu_sc as plsc

_N = 50000
_E = 800000
_C = 16
_K = 16
_D = 3
_NP = 50176
_CHT = 6272
_GN = _NP // _CHT
_NTASK = 51
_CHE = 4096
_EP = 802816
_NCH = _EP // _CHE
_PKPAD = -1018167296

_F32 = jnp.float32


def _erf(x):
    a1, a2, a3, a4, a5 = 0.254829592, -0.284496736, 1.421413741, -1.453152027, 1.061405429
    p = 0.3275911
    s = jnp.sign(x)
    z = jnp.abs(x)
    t = 1.0 / (1.0 + p * z)
    poly = ((((a5 * t + a4) * t + a3) * t + a2) * t + a1) * t
    return s * (1.0 - poly * jnp.exp(-z * z))


def _gelu(x):
    return 0.5 * x * (1.0 + _erf(x * 0.7071067811865476))


_P1 = 66


def _sc_unit(tid, piece, clo, chi, h_hbm, pk_hbm, egw_hbm, a_hbm, a2_hbm,
             fn_v, acc_v, pb0, eb0, pb1, eb1, sem0, sem1):
    d = tid // 17
    c = tid - d * 17
    pltpu.sync_copy(h_hbm.at[c], fn_v)

    def zbody(i, carry):
        acc_v[pl.ds(i * 16, 16)] = jnp.zeros((16,), _F32)
        return carry
    lax.fori_loop(0, _NP // 16, zbody, 0)

    ebase = d * _EP

    def start(ci, pb, eb, sem):
        off = ci * _CHE
        pltpu.async_copy(pk_hbm.at[pl.ds(off, _CHE)], pb, sem)
        pltpu.async_copy(egw_hbm.at[pl.ds(ebase + off, _CHE)], eb, sem)

    def wait2(pb, eb, sem):
        pltpu.make_async_copy(pk_hbm.at[pl.ds(0, _CHE)], pb, sem).wait()
        pltpu.make_async_copy(egw_hbm.at[pl.ds(0, _CHE)], eb, sem).wait()

    def process(pb, eb):
        @plsc.parallel_loop(0, _CHE, 16, unroll=32)
        def gbody(o):
            pk = pb[pl.ds(o, 16)]
            s = pk & 0xFFFF
            t = lax.shift_right_logical(pk, 16)
            w = eb[pl.ds(o, 16)]
            vals = plsc.load_gather(fn_v, [s])
            plsc.addupdate_scatter(acc_v, [t], vals * w)

    start(clo, pb0, eb0, sem0)
    start(clo + 1, pb1, eb1, sem1)

    def cbody(i, carry):
        ci = clo + i * 2
        wait2(pb0, eb0, sem0)
        process(pb0, eb0)

        @pl.when(ci + 2 < chi)
        def _():
            start(ci + 2, pb0, eb0, sem0)

        wait2(pb1, eb1, sem1)
        process(pb1, eb1)

        @pl.when(ci + 3 < chi)
        def _():
            start(ci + 3, pb1, eb1, sem1)
        return carry
    lax.fori_loop(0, (chi - clo) // 2, cbody, 0)

    @pl.when(piece == 0)
    def _():
        pltpu.sync_copy(acc_v, a_hbm.at[tid])

    @pl.when(piece == 1)
    def _():
        pltpu.sync_copy(acc_v, a2_hbm.at[tid - 32])

    @pl.when(piece == 2)
    def _():
        pltpu.sync_copy(acc_v, a2_hbm.at[tid - 13])


@functools.partial(
    pl.kernel,
    out_type=[jax.ShapeDtypeStruct((_NTASK, _NP), _F32),
              jax.ShapeDtypeStruct((38, _NP), _F32)],
    mesh=plsc.VectorSubcoreMesh(core_axis_name="c", subcore_axis_name="s",
                                num_cores=2, num_subcores=16),
    compiler_params=pltpu.CompilerParams(needs_layout_passes=False,
                                         use_tc_tiling_on_sc=True),
    scratch_types=[
        pltpu.VMEM((_NP,), _F32),
        pltpu.VMEM((_NP,), _F32),
        pltpu.VMEM((_CHE,), jnp.int32),
        pltpu.VMEM((_CHE,), _F32),
        pltpu.VMEM((_CHE,), jnp.int32),
        pltpu.VMEM((_CHE,), _F32),
        pltpu.SemaphoreType.DMA,
        pltpu.SemaphoreType.DMA,
    ],
)
def _sc_grad(h_hbm, pk_hbm, egw_hbm, a_hbm, a2_hbm,
             fn_v, acc_v, pb0, eb0, pb1, eb1, sem0, sem1):
    wid = lax.axis_index("s") * 2 + lax.axis_index("c")
    rest = [fn_v, acc_v, pb0, eb0, pb1, eb1, sem0, sem1]

    _sc_unit(wid, 0, 0, _NCH, h_hbm, pk_hbm, egw_hbm, a_hbm, a2_hbm, *rest)

    def split_unit(j):
        tid = 32 + j // 3
        piece = j - (j // 3) * 3
        clo = piece * _P1
        chi = jnp.minimum(clo + _P1, _NCH)
        _sc_unit(tid, piece, clo, chi, h_hbm, pk_hbm, egw_hbm, a_hbm, a2_hbm,
                 *rest)

    split_unit(wid)

    @pl.when(wid < 25)
    def _():
        split_unit(wid + 32)


def _full(shape):
    return pl.BlockSpec(shape, lambda *g: tuple(0 for _ in shape))


def _chunk(rows):
    return pl.BlockSpec((rows, _CHT), lambda *g: (0, g[-1]))


def _k1a_body(x_ref, fc0w_ref, fc0b_ref, h_ref):
    h = lax.dot_general(fc0w_ref[...], x_ref[...], (((1,), (0,)), ((), ())),
                        preferred_element_type=_F32) + fc0b_ref[...]
    h_ref[...] = jnp.concatenate(
        [h, jnp.ones((1, _CHT), _F32), jnp.zeros((7, _CHT), _F32)], axis=0)


def _k1a(x_T, fc0_w, fc0_b):
    return pl.pallas_call(
        _k1a_body,
        grid=(_GN,),
        in_specs=[_chunk(8), _full((16, 8)), _full((16, 1))],
        out_specs=_chunk(24),
        out_shape=jax.ShapeDtypeStruct((24, _NP), _F32),
    )(x_T, fc0_w, fc0_b)


def _k1b_body(nodes_ref, nwm_ref, mm_ref,
              bc_ref, bs_ref, wbc_ref, wbs_ref):
    t = lax.dot_general(mm_ref[...], nodes_ref[...], (((1,), (0,)), ((), ())),
                        preferred_element_type=_F32)
    bc = jnp.cos(t)
    bs = jnp.sin(t)
    wv = nwm_ref[0:1, :] * nwm_ref[1:2, :]
    bc_ref[...] = bc
    bs_ref[...] = bs
    wbc_ref[...] = bc * wv
    wbs_ref[...] = bs * wv


def _k1b(nodes_T, nwm, mm):
    return pl.pallas_call(
        _k1b_body,
        grid=(_GN,),
        in_specs=[_chunk(8), _chunk(8), _full((16, 8))],
        out_specs=[_chunk(16)] * 4,
        out_shape=[jax.ShapeDtypeStruct((16, _NP), _F32)] * 4,
    )(nodes_T, nwm, mm)


def _k2_body(h_ref, wbc_ref, wbs_ref, sc_ref, ss_ref):
    @pl.when(pl.program_id(0) == 0)
    def _():
        sc_ref[...] = jnp.zeros_like(sc_ref)
        ss_ref[...] = jnp.zeros_like(ss_ref)
    h = h_ref[0:16, :]
    sc_ref[...] += lax.dot_general(h, wbc_ref[...], (((1,), (1,)), ((), ())),
                                   preferred_element_type=_F32)
    ss_ref[...] += lax.dot_general(h, wbs_ref[...], (((1,), (1,)), ((), ())),
                                   preferred_element_type=_F32)


def _k2(h_ext, wbc, wbs):
    return pl.pallas_call(
        _k2_body,
        grid=(_GN,),
        in_specs=[_chunk(24), _chunk(16), _chunk(16)],
        out_specs=[pl.BlockSpec((16, 16), lambda j: (0, 0))] * 2,
        out_shape=[jax.ShapeDtypeStruct((16, 16), _F32)] * 2,
    )(h_ext, wbc, wbs)


def _k3_body(sc_ref, ss_ref, wc_ref, ws_ref, wsw_ref, wsb_ref,
             gwsp_ref, gwsd_ref, gwsb_ref, w1_ref, b1_ref, w2_ref, b2_ref,
             h_ref, bc_ref, bs_ref, a_ref, a2_ref, out_ref, *, last):
    h = h_ref[0:16, :]
    Sc = sc_ref[...]
    Ss = ss_ref[...]
    wc = wc_ref[...]
    ws = ws_ref[...]
    ein = lambda X, W: jnp.sum(X[:, None, :] * W, axis=0)
    f_c = ein(Sc, wc) + ein(Ss, ws)
    f_s = ein(Sc, ws) - ein(Ss, wc)
    x1 = 2.0 * (lax.dot_general(f_c, bc_ref[...], (((1,), (0,)), ((), ())),
                                preferred_element_type=_F32)
                - lax.dot_general(f_s, bs_ref[...], (((1,), (0,)), ((), ())),
                                  preferred_element_type=_F32))
    x2 = lax.dot_general(wsw_ref[...], h, (((1,), (0,)), ((), ())),
                         preferred_element_type=_F32) + wsb_ref[...]
    aeff = jnp.concatenate(
        [a_ref[0:32, :],
         a_ref[32:_NTASK, :] + a2_ref[0:19, :] + a2_ref[19:38, :]],
        axis=0)
    x3 = lax.dot_general(gwsp_ref[...], aeff, (((1,), (0,)), ((), ())),
                         preferred_element_type=_F32) + gwsb_ref[...]
    for dd in range(_D):
        corr = lax.dot_general(gwsd_ref[dd], h, (((1,), (0,)), ((), ())),
                               preferred_element_type=_F32)
        x3 = x3 - corr * aeff[dd * 17 + 16:dd * 17 + 17, :]
    hn = x1 + x2 + x3
    if last:
        z = _gelu(lax.dot_general(w1_ref[...], hn, (((1,), (0,)), ((), ())),
                                  preferred_element_type=_F32) + b1_ref[...])
        out_ref[...] = lax.dot_general(w2_ref[...], z, (((1,), (0,)), ((), ())),
                                       preferred_element_type=_F32) + b2_ref[...]
    else:
        out_ref[...] = jnp.concatenate(
            [_gelu(hn), jnp.ones((1, _CHT), _F32),
             jnp.zeros((7, _CHT), _F32)], axis=0)


def _k3(Sc, Ss, wc, ws, wsw, wsb, gwsp, gwsd, gwsb, w1, b1, w2, b2,
        h_ext, bc, bs, A, A2, last):
    out_rows = 1 if last else 24
    return pl.pallas_call(
        functools.partial(_k3_body, last=last),
        grid=(_GN,),
        in_specs=[_full((16, 16)), _full((16, 16)),
                  _full((16, 16, 16)), _full((16, 16, 16)),
                  _full((16, 16)), _full((16, 1)),
                  _full((16, _NTASK)), _full((3, 16, 16)), _full((16, 1)),
                  _full((128, 16)), _full((128, 1)), _full((1, 128)),
                  _full((1, 1)),
                  _chunk(24), _chunk(16), _chunk(16),
                  _chunk(_NTASK), _chunk(38)],
        out_specs=_chunk(out_rows),
        out_shape=jax.ShapeDtypeStruct((out_rows, _NP), _F32),
    )(Sc, Ss, wc, ws, wsw, wsb, gwsp, gwsd, gwsb, w1, b1, w2, b2,
      h_ext, bc, bs, A, A2)


def kernel(x, node_mask, nodes, node_weights, directed_edges,
           edge_gradient_weights, sp_L, modes, fc0_w, fc0_b, ws_w, ws_b,
           gws_w, gws_b, spec_wc, spec_ws, spec_w0, fc1_w, fc1_b, fc2_w,
           fc2_b):
    pad = _NP - _N
    mm = jnp.pad((modes * sp_L[None, :, :])[:, :, 0], ((0, 0), (0, 5)))
    nodes_T = jnp.pad(nodes[0].T, ((0, 5), (0, pad)))
    x_T = jnp.pad(x[0].T, ((0, 4), (0, pad)))
    nwm = jnp.pad(jnp.concatenate([node_weights[0].T, node_mask[0].T], 0),
                  ((0, 6), (0, pad)))
    fc0_w8 = jnp.pad(fc0_w, ((0, 0), (0, 4)))
    src = directed_edges[0, :, 1]
    tgt = directed_edges[0, :, 0]
    pk = jnp.pad((tgt << 16) | src, (0, _EP - _E), constant_values=_PKPAD)
    egw_flat = jnp.pad(edge_gradient_weights[0].T,
                       ((0, 0), (0, _EP - _E))).reshape(-1)

    L = ws_w.shape[0]
    g4 = gws_w.reshape(L, _C, _C, _D)
    gws_d = jnp.transpose(g4, (0, 3, 1, 2))
    gp = jnp.transpose(g4, (0, 1, 3, 2))
    gws_p = jnp.concatenate([gp, jnp.zeros((L, _C, _D, 1), _F32)],
                            axis=3).reshape(L, _C, _NTASK)

    h_ext = _k1a(x_T, fc0_w8, fc0_b.reshape(16, 1))
    bc, bs, wbc, wbs = _k1b(nodes_T, nwm, mm)

    out = None
    for i in range(L):
        A, A2 = _sc_grad(h_ext, pk, egw_flat)
        Sc, Ss = _k2(h_ext, wbc, wbs)
        res = _k3(Sc, Ss, spec_wc[i][:, :, :, 0], spec_ws[i][:, :, :, 0],
                  ws_w[i], ws_b[i].reshape(16, 1), gws_p[i], gws_d[i],
                  gws_b[i].reshape(16, 1), fc1_w, fc1_b.reshape(128, 1),
                  fc2_w, fc2_b.reshape(1, 1),
                  h_ext, bc, bs, A, A2, last=(i == L - 1))
        if i == L - 1:
            out = res
        else:
            h_ext = res

    return out[:, :_N][:, :, None]
